# baseline (speedup 1.0000x reference)
import functools

import jax
import jax.numpy as jnp
from jax import lax
from jax.experimental import pallas as pl
from jax.experimental.pallas import tpu as pltpu
from jax.experimental.pallas import tpu_sc as plsc

DIM = 64
SEQ = 200
NUM_CLASSES = 128
NC = 2
NS = 16
NW = NC * NS

_C0, _C1 = 104, 96
NBUF = 2


def _pool_body(idx_hbm, ne_hbm, table_hbm, out_hbm,
               idx_v, ne_v, rows_v, out_v, sem0, sem1):
  batch_dim = out_hbm.shape[0]
  bpw = batch_dim // (NW * DIM)
  wid = lax.axis_index("s") * NC + lax.axis_index("c")
  base = wid * bpw * SEQ
  sems = (sem0, sem1)

  pltpu.sync_copy(idx_hbm.at[pl.ds(base, bpw * SEQ)], idx_v)
  pltpu.sync_copy(ne_hbm.at[pl.ds(wid * bpw * 16, bpw * 16)], ne_v)

  def start_row(i, b):
    pltpu.make_async_copy(
        table_hbm.at[idx_v.at[pl.ds(i * SEQ, _C0)]],
        rows_v.at[b, pl.ds(0, _C0)], sems[b]).start()
    pltpu.make_async_copy(
        table_hbm.at[idx_v.at[pl.ds(i * SEQ + _C0, _C1)]],
        rows_v.at[b, pl.ds(_C0, _C1)], sems[b]).start()

  def wait_row(b):
    pltpu.make_async_copy(table_hbm.at[pl.ds(0, SEQ)],
                          rows_v.at[b], sems[b]).wait()

  def accum_row(i, b):
    ne = jnp.max(ne_v[pl.ds(i * 16, 16)])

    def make_body(off):
      def jbody(j, carry):
        a = list(carry)
        for k in range(4):
          a[k] = a[k] + rows_v[b, j, pl.ds(off + 16 * k, 16)]
        return tuple(a)
      return jbody

    zero4 = tuple(jnp.zeros((16,), jnp.float32) for _ in range(4))
    acc = lax.fori_loop(0, ne, make_body(0), zero4)
    acc = lax.fori_loop(ne, SEQ, make_body(DIM), acc)
    for k in range(4):
      out_v[pl.ds(pl.multiple_of(i * DIM + 16 * k, 16), 16)] = acc[k]

  for b in range(NBUF):
    start_row(b, b)

  def gbody(t, _):
    for b in range(NBUF):
      i = t * NBUF + b
      wait_row(b)
      accum_row(i, b)
      start_row(i + NBUF, b)
    return 0

  lax.fori_loop(0, (bpw - NBUF) // NBUF, gbody, 0)
  for b in range(NBUF):
    wait_row(b)
    accum_row(bpw - NBUF + b, b)

  pltpu.sync_copy(out_v, out_hbm.at[pl.ds(wid * bpw * DIM, bpw * DIM)])


def _pool(idx_half, ne_rep, table2):
  batch = idx_half.shape[0] // SEQ
  bpw = batch // NW
  mesh = plsc.VectorSubcoreMesh(core_axis_name="c", subcore_axis_name="s")
  k = functools.partial(
      pl.kernel,
      out_type=jax.ShapeDtypeStruct((batch * DIM,), jnp.float32),
      mesh=mesh,
      scratch_types=[
          pltpu.VMEM((bpw * SEQ,), jnp.int32),
          pltpu.VMEM((bpw * 16,), jnp.int32),
          pltpu.VMEM((NBUF, SEQ, 2 * DIM), jnp.float32),
          pltpu.VMEM((bpw * DIM,), jnp.float32),
          pltpu.SemaphoreType.DMA,
          pltpu.SemaphoreType.DMA,
      ],
      compiler_params=pltpu.CompilerParams(use_tc_tiling_on_sc=True,
                                           needs_layout_passes=False),
  )(_pool_body)
  return k(idx_half, ne_rep, table2)


def _mlp_body(x_ref, w1_ref, b1_ref, w2_ref, b2_ref, out_ref):
  x = x_ref[:]
  h = jnp.tanh(
      lax.dot_general(x, w1_ref[:], (((1,), (1,)), ((), ())),
                      preferred_element_type=jnp.float32) + b1_ref[:])
  out_ref[:] = lax.dot_general(
      h, w2_ref[:], (((1,), (1,)), ((), ())),
      preferred_element_type=jnp.float32) + b2_ref[:]


def _mlp(pooled, W1, b1, W2, b2):
  batch = pooled.shape[0]
  blk = 1024
  return pl.pallas_call(
      _mlp_body,
      grid=(batch // blk,),
      in_specs=[
          pl.BlockSpec((blk, DIM), lambda i: (i, 0)),
          pl.BlockSpec((DIM, DIM), lambda i: (0, 0)),
          pl.BlockSpec((1, DIM), lambda i: (0, 0)),
          pl.BlockSpec((NUM_CLASSES, DIM), lambda i: (0, 0)),
          pl.BlockSpec((1, NUM_CLASSES), lambda i: (0, 0)),
      ],
      out_specs=pl.BlockSpec((blk, NUM_CLASSES), lambda i: (i, 0)),
      out_shape=jax.ShapeDtypeStruct((batch, NUM_CLASSES), jnp.float32),
  )(pooled, W1, b1.reshape(1, DIM), W2, b2.reshape(1, NUM_CLASSES))


def kernel(word_ids, table, W1, b1, W2, b2):
  ids = word_ids.astype(jnp.int32)
  key = ((ids & 1) << 19) | (ids >> 1)
  skey = jnp.sort(key, axis=1)
  idx_half = (skey & ((1 << 19) - 1)).reshape(-1)
  ne = SEQ - (ids & 1).sum(axis=1, dtype=jnp.int32)
  ne_rep = jnp.repeat(ne, 16)
  table2 = table.reshape(table.shape[0] // 2, 2 * DIM)
  pooled = _pool(idx_half, ne_rep, table2).reshape(ids.shape[0], DIM)
  return _mlp(pooled, W1, b1, W2, b2)


# speedup vs baseline: 1.1258x; 1.1258x over previous
import functools

import jax
import jax.numpy as jnp
from jax import lax
from jax.experimental import pallas as pl
from jax.experimental.pallas import tpu as pltpu
from jax.experimental.pallas import tpu_sc as plsc

DIM = 64
SEQ = 200
SEQP = 208
NUM_CLASSES = 128
NC = 2
NS = 16
NW = NC * NS

_C0, _C1 = 104, 96
NBUF = 2

_DCOL = 2048
_DROW = _DCOL // 2


def _detile_body(x_ref, o_ref):
  x = x_ref[:]
  o_ref[:, 0:DIM] = x[:, 0:_DROW].T
  o_ref[:, DIM:2 * DIM] = x[:, _DROW:_DCOL].T


def _detile(table):
  vocab = table.shape[0]
  grid = (vocab + _DCOL - 1) // _DCOL
  return pl.pallas_call(
      _detile_body,
      grid=(grid,),
      in_specs=[pl.BlockSpec((DIM, _DCOL), lambda i: (0, i))],
      out_specs=pl.BlockSpec((_DROW, 2 * DIM), lambda i: (i, 0)),
      out_shape=jax.ShapeDtypeStruct((grid * _DROW, 2 * DIM), jnp.float32),
  )(table.T)


def _pool_body(idx_hbm, off_hbm, table_hbm, out_hbm,
               idx_v, off_v, rows_v, out_v, sem0, sem1):
  batch_dim = out_hbm.shape[0]
  bpw = batch_dim // (NW * DIM)
  wid = lax.axis_index("s") * NC + lax.axis_index("c")
  sems = (sem0, sem1)

  pltpu.sync_copy(idx_hbm.at[pl.ds(wid * bpw * SEQ, bpw * SEQ)], idx_v)
  pltpu.sync_copy(off_hbm.at[pl.ds(wid * bpw * SEQP, bpw * SEQP)], off_v)

  def start_row(i, b):
    pltpu.make_async_copy(
        table_hbm.at[idx_v.at[pl.ds(i * SEQ, _C0)]],
        rows_v.at[b, pl.ds(0, _C0)], sems[b]).start()
    pltpu.make_async_copy(
        table_hbm.at[idx_v.at[pl.ds(i * SEQ + _C0, _C1)]],
        rows_v.at[b, pl.ds(_C0, _C1)], sems[b]).start()

  def wait_row(b):
    pltpu.make_async_copy(table_hbm.at[pl.ds(0, SEQ)],
                          rows_v.at[b], sems[b]).wait()

  def accum_16(i, b, t, acc, n_js):
    par16 = off_v[pl.ds(pl.multiple_of(i * SEQP + 16 * t, 16), 16)]
    a = list(acc)
    for u in range(n_js):
      j = 16 * t + u
      m = lax.gather(
          par16, jnp.full((16, 1), u, jnp.int32),
          lax.GatherDimensionNumbers(offset_dims=(), collapsed_slice_dims=(0,),
                                     start_index_map=(0,)),
          (1,), mode=lax.GatherScatterMode.PROMISE_IN_BOUNDS) > 0
      for k in range(4):
        lo = rows_v[b, j, pl.ds(16 * k, 16)]
        hi = rows_v[b, j, pl.ds(DIM + 16 * k, 16)]
        a[k] = a[k] + jnp.where(m, hi, lo)
    return tuple(a)

  def accum_row(i, b):
    def tbody(t, carry):
      return accum_16(i, b, t, carry, 16)
    acc = lax.fori_loop(
        0, SEQ // 16, tbody,
        tuple(jnp.zeros((16,), jnp.float32) for _ in range(4)))
    acc = accum_16(i, b, SEQ // 16, acc, SEQ - 16 * (SEQ // 16))
    for k in range(4):
      out_v[pl.ds(pl.multiple_of(i * DIM + 16 * k, 16), 16)] = acc[k]

  for b in range(NBUF):
    start_row(b, b)

  def gbody(t, _):
    for b in range(NBUF):
      i = t * NBUF + b
      wait_row(b)
      accum_row(i, b)
      start_row(i + NBUF, b)
    return 0

  lax.fori_loop(0, (bpw - NBUF) // NBUF, gbody, 0)
  for b in range(NBUF):
    wait_row(b)
    accum_row(bpw - NBUF + b, b)

  pltpu.sync_copy(out_v, out_hbm.at[pl.ds(wid * bpw * DIM, bpw * DIM)])


def _pool(idx_half, off64, table2):
  batch = idx_half.shape[0] // SEQ
  bpw = batch // NW
  mesh = plsc.VectorSubcoreMesh(core_axis_name="c", subcore_axis_name="s")
  k = functools.partial(
      pl.kernel,
      out_type=jax.ShapeDtypeStruct((batch * DIM,), jnp.float32),
      mesh=mesh,
      scratch_types=[
          pltpu.VMEM((bpw * SEQ,), jnp.int32),
          pltpu.VMEM((bpw * SEQP,), jnp.int32),
          pltpu.VMEM((NBUF, SEQ, 2 * DIM), jnp.float32),
          pltpu.VMEM((bpw * DIM,), jnp.float32),
          pltpu.SemaphoreType.DMA,
          pltpu.SemaphoreType.DMA,
      ],
      compiler_params=pltpu.CompilerParams(use_tc_tiling_on_sc=True,
                                           needs_layout_passes=False),
  )(_pool_body)
  return k(idx_half, off64, table2)


def _mlp_body(x_ref, w1_ref, b1_ref, w2_ref, b2_ref, out_ref):
  x = x_ref[:]
  h = jnp.tanh(
      lax.dot_general(x, w1_ref[:], (((1,), (1,)), ((), ())),
                      preferred_element_type=jnp.float32) + b1_ref[:])
  out_ref[:] = lax.dot_general(
      h, w2_ref[:], (((1,), (1,)), ((), ())),
      preferred_element_type=jnp.float32) + b2_ref[:]


def _mlp(pooled, W1, b1, W2, b2):
  batch = pooled.shape[0]
  blk = 1024
  return pl.pallas_call(
      _mlp_body,
      grid=(batch // blk,),
      in_specs=[
          pl.BlockSpec((blk, DIM), lambda i: (i, 0)),
          pl.BlockSpec((DIM, DIM), lambda i: (0, 0)),
          pl.BlockSpec((1, DIM), lambda i: (0, 0)),
          pl.BlockSpec((NUM_CLASSES, DIM), lambda i: (0, 0)),
          pl.BlockSpec((1, NUM_CLASSES), lambda i: (0, 0)),
      ],
      out_specs=pl.BlockSpec((blk, NUM_CLASSES), lambda i: (i, 0)),
      out_shape=jax.ShapeDtypeStruct((batch, NUM_CLASSES), jnp.float32),
  )(pooled, W1, b1.reshape(1, DIM), W2, b2.reshape(1, NUM_CLASSES))


def kernel(word_ids, table, W1, b1, W2, b2):
  ids = word_ids.astype(jnp.int32)
  idx_half = (((ids >> 11) << 10) | (ids & 1023)).reshape(-1)
  off64 = jnp.pad(((ids >> 10) & 1) << 6,
                  ((0, 0), (0, SEQP - SEQ))).reshape(-1)
  table2 = _detile(table)
  pooled = _pool(idx_half, off64, table2).reshape(ids.shape[0], DIM)
  return _mlp(pooled, W1, b1, W2, b2)


# speedup vs baseline: 1.3425x; 1.1925x over previous
import functools

import jax
import jax.numpy as jnp
from jax import lax
from jax.experimental import pallas as pl
from jax.experimental.pallas import tpu as pltpu
from jax.experimental.pallas import tpu_sc as plsc

DIM = 64
SEQ = 200
SEQP = 208
NUM_CLASSES = 128
NC = 2
NS = 16
NW = NC * NS

_C0, _C1 = 104, 96
NBUF = 2

_DCOL = 4096
_DROW = _DCOL // 2


def _detile_body(x_ref, eye_ref, o_ref):
  x = x_ref[:]
  eye = eye_ref[:]
  o_ref[:, 0:DIM] = lax.dot_general(
      x[:, 0:_DROW], eye, (((0,), (0,)), ((), ())),
      preferred_element_type=jnp.float32)
  o_ref[:, DIM:2 * DIM] = lax.dot_general(
      x[:, _DROW:_DCOL], eye, (((0,), (0,)), ((), ())),
      preferred_element_type=jnp.float32)


def _detile(table):
  vocab = table.shape[0]
  grid = (vocab + _DCOL - 1) // _DCOL
  return pl.pallas_call(
      _detile_body,
      grid=(grid,),
      in_specs=[
          pl.BlockSpec((DIM, _DCOL), lambda i: (0, i)),
          pl.BlockSpec((DIM, DIM), lambda i: (0, 0)),
      ],
      out_specs=pl.BlockSpec((_DROW, 2 * DIM), lambda i: (i, 0)),
      out_shape=jax.ShapeDtypeStruct((grid * _DROW, 2 * DIM), jnp.float32),
  )(table.T, jnp.eye(DIM, dtype=jnp.float32))


def _pool_body(idx_hbm, off_hbm, table_hbm, out_hbm,
               idx_v, off_v, rows_v, out_v, sem0, sem1):
  batch_dim = out_hbm.shape[0]
  bpw = batch_dim // (NW * DIM)
  wid = lax.axis_index("s") * NC + lax.axis_index("c")
  sems = (sem0, sem1)

  pltpu.sync_copy(idx_hbm.at[pl.ds(wid * bpw * SEQ, bpw * SEQ)], idx_v)
  pltpu.sync_copy(off_hbm.at[pl.ds(wid * bpw * SEQP, bpw * SEQP)], off_v)

  def start_row(i, b):
    pltpu.make_async_copy(
        table_hbm.at[idx_v.at[pl.ds(i * SEQ, _C0)]],
        rows_v.at[b, pl.ds(0, _C0)], sems[b]).start()
    pltpu.make_async_copy(
        table_hbm.at[idx_v.at[pl.ds(i * SEQ + _C0, _C1)]],
        rows_v.at[b, pl.ds(_C0, _C1)], sems[b]).start()

  def wait_row(b):
    pltpu.make_async_copy(table_hbm.at[pl.ds(0, SEQ)],
                          rows_v.at[b], sems[b]).wait()

  def accum_16(i, b, t, acc, n_js):
    par16 = off_v[pl.ds(pl.multiple_of(i * SEQP + 16 * t, 16), 16)]
    a = list(acc)
    for u in range(n_js):
      j = 16 * t + u
      m = lax.gather(
          par16, jnp.full((16, 1), u, jnp.int32),
          lax.GatherDimensionNumbers(offset_dims=(), collapsed_slice_dims=(0,),
                                     start_index_map=(0,)),
          (1,), mode=lax.GatherScatterMode.PROMISE_IN_BOUNDS) > 0
      for k in range(4):
        lo = rows_v[b, j, pl.ds(16 * k, 16)]
        hi = rows_v[b, j, pl.ds(DIM + 16 * k, 16)]
        a[k] = a[k] + jnp.where(m, hi, lo)
    return tuple(a)

  def accum_row(i, b):
    def tbody(t, carry):
      return accum_16(i, b, t, carry, 16)
    acc = lax.fori_loop(
        0, SEQ // 16, tbody,
        tuple(jnp.zeros((16,), jnp.float32) for _ in range(4)))
    acc = accum_16(i, b, SEQ // 16, acc, SEQ - 16 * (SEQ // 16))
    for k in range(4):
      out_v[pl.ds(pl.multiple_of(i * DIM + 16 * k, 16), 16)] = acc[k]

  for b in range(NBUF):
    start_row(b, b)

  def gbody(t, _):
    for b in range(NBUF):
      i = t * NBUF + b
      wait_row(b)
      accum_row(i, b)
      start_row(i + NBUF, b)
    return 0

  lax.fori_loop(0, (bpw - NBUF) // NBUF, gbody, 0)
  for b in range(NBUF):
    wait_row(b)
    accum_row(bpw - NBUF + b, b)

  pltpu.sync_copy(out_v, out_hbm.at[pl.ds(wid * bpw * DIM, bpw * DIM)])


def _pool(idx_half, off64, table2):
  batch = idx_half.shape[0] // SEQ
  bpw = batch // NW
  mesh = plsc.VectorSubcoreMesh(core_axis_name="c", subcore_axis_name="s")
  k = functools.partial(
      pl.kernel,
      out_type=jax.ShapeDtypeStruct((batch * DIM,), jnp.float32),
      mesh=mesh,
      scratch_types=[
          pltpu.VMEM((bpw * SEQ,), jnp.int32),
          pltpu.VMEM((bpw * SEQP,), jnp.int32),
          pltpu.VMEM((NBUF, SEQ, 2 * DIM), jnp.float32),
          pltpu.VMEM((bpw * DIM,), jnp.float32),
          pltpu.SemaphoreType.DMA,
          pltpu.SemaphoreType.DMA,
      ],
      compiler_params=pltpu.CompilerParams(use_tc_tiling_on_sc=True,
                                           needs_layout_passes=False),
  )(_pool_body)
  return k(idx_half, off64, table2)


def _mlp_body(x_ref, w1_ref, b1_ref, w2_ref, b2_ref, out_ref):
  x = x_ref[:]
  h = jnp.tanh(
      lax.dot_general(x, w1_ref[:], (((1,), (1,)), ((), ())),
                      preferred_element_type=jnp.float32) + b1_ref[:])
  out_ref[:] = lax.dot_general(
      h, w2_ref[:], (((1,), (1,)), ((), ())),
      preferred_element_type=jnp.float32) + b2_ref[:]


def _mlp(pooled, W1, b1, W2, b2):
  batch = pooled.shape[0]
  blk = 1024
  return pl.pallas_call(
      _mlp_body,
      grid=(batch // blk,),
      in_specs=[
          pl.BlockSpec((blk, DIM), lambda i: (i, 0)),
          pl.BlockSpec((DIM, DIM), lambda i: (0, 0)),
          pl.BlockSpec((1, DIM), lambda i: (0, 0)),
          pl.BlockSpec((NUM_CLASSES, DIM), lambda i: (0, 0)),
          pl.BlockSpec((1, NUM_CLASSES), lambda i: (0, 0)),
      ],
      out_specs=pl.BlockSpec((blk, NUM_CLASSES), lambda i: (i, 0)),
      out_shape=jax.ShapeDtypeStruct((batch, NUM_CLASSES), jnp.float32),
  )(pooled, W1, b1.reshape(1, DIM), W2, b2.reshape(1, NUM_CLASSES))


def kernel(word_ids, table, W1, b1, W2, b2):
  ids = word_ids.astype(jnp.int32)
  idx_half = ((ids // _DCOL) * _DROW + (ids % _DROW)).reshape(-1)
  off64 = jnp.pad(((ids // _DROW) & 1) << 6,
                  ((0, 0), (0, SEQP - SEQ))).reshape(-1)
  table2 = _detile(table)
  pooled = _pool(idx_half, off64, table2).reshape(ids.shape[0], DIM)
  return _mlp(pooled, W1, b1, W2, b2)


# speedup vs baseline: 2.1840x; 1.6268x over previous
import functools

import jax
import jax.numpy as jnp
from jax import lax
from jax.experimental import pallas as pl
from jax.experimental.pallas import tpu as pltpu
from jax.experimental.pallas import tpu_sc as plsc

DIM = 64
SEQ = 200
NUM_CLASSES = 128
NC = 2
NS = 16
NW = NC * NS

_C0, _C1 = 104, 96
NBUF = 4

_DCOL = 8192
_DROW = _DCOL // 2


def _detile_body(x_ref, eye_ref, o_ref):
  x = x_ref[:]
  eye = eye_ref[:]
  o_ref[:, 0:DIM] = lax.dot_general(
      x[:, 0:_DROW], eye, (((0,), (0,)), ((), ())),
      preferred_element_type=jnp.float32)
  o_ref[:, DIM:2 * DIM] = lax.dot_general(
      x[:, _DROW:_DCOL], eye, (((0,), (0,)), ((), ())),
      preferred_element_type=jnp.float32)


def _detile(table):
  vocab = table.shape[0]
  grid = (vocab + _DCOL - 1) // _DCOL
  return pl.pallas_call(
      _detile_body,
      grid=(grid,),
      in_specs=[
          pl.BlockSpec((DIM, _DCOL), lambda i: (0, i)),
          pl.BlockSpec((DIM, DIM), lambda i: (0, 0)),
      ],
      out_specs=pl.BlockSpec((_DROW, 2 * DIM), lambda i: (i, 0)),
      out_shape=jax.ShapeDtypeStruct((grid * _DROW, 2 * DIM), jnp.float32),
  )(table.T, jnp.eye(DIM, dtype=jnp.float32))


def _pool_body(ids_hbm, table_hbm, out_hbm, idx_v, rows_v, out_v, *sems):
  batch = out_hbm.shape[0]
  bpw = batch // NW
  wid = lax.axis_index("s") * NC + lax.axis_index("c")
  base = wid * bpw

  pltpu.sync_copy(ids_hbm.at[pl.ds(base, bpw)], idx_v)

  def start_row(i, b):
    pltpu.make_async_copy(
        table_hbm.at[idx_v.at[i, pl.ds(0, _C0)]],
        rows_v.at[b, pl.ds(0, _C0)], sems[b]).start()
    pltpu.make_async_copy(
        table_hbm.at[idx_v.at[i, pl.ds(_C0, _C1)]],
        rows_v.at[b, pl.ds(_C0, _C1)], sems[b]).start()

  def wait_row(b):
    pltpu.make_async_copy(table_hbm.at[pl.ds(0, SEQ)],
                          rows_v.at[b], sems[b]).wait()

  def accum_row(i, b):
    def jbody(jj, carry):
      a = list(carry)
      j = jj * 4
      for u in range(4):
        for k in range(4):
          a[k] = a[k] + rows_v[b, j + u, pl.ds(16 * k, 16)]
      return tuple(a)
    acc = lax.fori_loop(
        0, SEQ // 4, jbody,
        tuple(jnp.zeros((16,), jnp.float32) for _ in range(4)))
    for k in range(4):
      out_v[i, pl.ds(16 * k, 16)] = acc[k]

  for b in range(NBUF):
    start_row(b, b)

  def gbody(t, _):
    for b in range(NBUF):
      i = t * NBUF + b
      wait_row(b)
      accum_row(i, b)
      start_row(i + NBUF, b)
    return 0

  lax.fori_loop(0, (bpw - NBUF) // NBUF, gbody, 0)
  for b in range(NBUF):
    wait_row(b)
    accum_row(bpw - NBUF + b, b)

  pltpu.sync_copy(out_v, out_hbm.at[pl.ds(base, bpw)])


def _pool(ids_mapped, table_flat):
  batch = ids_mapped.shape[0]
  bpw = batch // NW
  mesh = plsc.VectorSubcoreMesh(core_axis_name="c", subcore_axis_name="s")
  k = functools.partial(
      pl.kernel,
      out_type=jax.ShapeDtypeStruct((batch, DIM), jnp.float32),
      mesh=mesh,
      scratch_types=[
          pltpu.VMEM((bpw, SEQ), jnp.int32),
          pltpu.VMEM((NBUF, SEQ, DIM), jnp.float32),
          pltpu.VMEM((bpw, DIM), jnp.float32),
      ] + [pltpu.SemaphoreType.DMA] * NBUF,
      compiler_params=pltpu.CompilerParams(use_tc_tiling_on_sc=False),
  )(_pool_body)
  return k(ids_mapped, table_flat)


def _mlp_body(x_ref, w1_ref, b1_ref, w2_ref, b2_ref, out_ref):
  x = x_ref[:]
  h = jnp.tanh(
      lax.dot_general(x, w1_ref[:], (((1,), (1,)), ((), ())),
                      preferred_element_type=jnp.float32) + b1_ref[:])
  out_ref[:] = lax.dot_general(
      h, w2_ref[:], (((1,), (1,)), ((), ())),
      preferred_element_type=jnp.float32) + b2_ref[:]


def _mlp(pooled, W1, b1, W2, b2):
  batch = pooled.shape[0]
  blk = 1024
  return pl.pallas_call(
      _mlp_body,
      grid=(batch // blk,),
      in_specs=[
          pl.BlockSpec((blk, DIM), lambda i: (i, 0)),
          pl.BlockSpec((DIM, DIM), lambda i: (0, 0)),
          pl.BlockSpec((1, DIM), lambda i: (0, 0)),
          pl.BlockSpec((NUM_CLASSES, DIM), lambda i: (0, 0)),
          pl.BlockSpec((1, NUM_CLASSES), lambda i: (0, 0)),
      ],
      out_specs=pl.BlockSpec((blk, NUM_CLASSES), lambda i: (i, 0)),
      out_shape=jax.ShapeDtypeStruct((batch, NUM_CLASSES), jnp.float32),
  )(pooled, W1, b1.reshape(1, DIM), W2, b2.reshape(1, NUM_CLASSES))


def kernel(word_ids, table, W1, b1, W2, b2):
  ids = word_ids.astype(jnp.int32)
  ids_mapped = ((ids // _DCOL) * _DCOL + 2 * (ids % _DROW)
                + ((ids // _DROW) & 1))
  table2 = _detile(table)
  table_flat = table2.reshape(table2.shape[0] * 2, DIM)
  pooled = _pool(ids_mapped, table_flat)
  return _mlp(pooled, W1, b1, W2, b2)


# speedup vs baseline: 2.3673x; 1.0840x over previous
import functools

import jax
import jax.numpy as jnp
from jax import lax
from jax.experimental import pallas as pl
from jax.experimental.pallas import tpu as pltpu
from jax.experimental.pallas import tpu_sc as plsc

DIM = 64
SEQ = 200
NUM_CLASSES = 128
NC = 2
NS = 16
NW = NC * NS

_C0, _C1 = 104, 96
NBUF = 4

_DCOL = 8192
_DROW = _DCOL // 2


def _detile_body(x_ref, eye_ref, o_ref):
  x = x_ref[:].astype(jnp.bfloat16)
  eye = eye_ref[:]
  o_ref[:, 0:DIM] = lax.dot_general(
      x[:, 0:_DROW], eye, (((0,), (0,)), ((), ())),
      preferred_element_type=jnp.float32)
  o_ref[:, DIM:2 * DIM] = lax.dot_general(
      x[:, _DROW:_DCOL], eye, (((0,), (0,)), ((), ())),
      preferred_element_type=jnp.float32)


def _detile(table):
  vocab = table.shape[0]
  grid = (vocab + _DCOL - 1) // _DCOL
  return pl.pallas_call(
      _detile_body,
      grid=(grid,),
      in_specs=[
          pl.BlockSpec((DIM, _DCOL), lambda i: (0, i)),
          pl.BlockSpec((DIM, DIM), lambda i: (0, 0)),
      ],
      out_specs=pl.BlockSpec((_DROW, 2 * DIM), lambda i: (i, 0)),
      out_shape=jax.ShapeDtypeStruct((grid * _DROW, 2 * DIM), jnp.float32),
  )(table.T, jnp.eye(DIM, dtype=jnp.bfloat16))


def _pool_body(ids_hbm, table_hbm, out_hbm, idx_v, rows_v, out_v, *sems):
  batch = out_hbm.shape[0]
  bpw = batch // NW
  wid = lax.axis_index("s") * NC + lax.axis_index("c")
  base = wid * bpw

  pltpu.sync_copy(ids_hbm.at[pl.ds(base, bpw)], idx_v)

  def start_row(i, b):
    pltpu.make_async_copy(
        table_hbm.at[idx_v.at[i, pl.ds(0, _C0)]],
        rows_v.at[b, pl.ds(0, _C0)], sems[b]).start()
    pltpu.make_async_copy(
        table_hbm.at[idx_v.at[i, pl.ds(_C0, _C1)]],
        rows_v.at[b, pl.ds(_C0, _C1)], sems[b]).start()

  def wait_row(b):
    pltpu.make_async_copy(table_hbm.at[pl.ds(0, SEQ)],
                          rows_v.at[b], sems[b]).wait()

  def accum_row(i, b):
    def jbody(jj, carry):
      a = list(carry)
      j = jj * 4
      for u in range(4):
        for k in range(4):
          a[k] = a[k] + rows_v[b, j + u, pl.ds(16 * k, 16)]
      return tuple(a)
    acc = lax.fori_loop(
        0, SEQ // 4, jbody,
        tuple(jnp.zeros((16,), jnp.float32) for _ in range(4)))
    for k in range(4):
      out_v[i, pl.ds(16 * k, 16)] = acc[k]

  for b in range(NBUF):
    start_row(b, b)

  def gbody(t, _):
    for b in range(NBUF):
      i = t * NBUF + b
      wait_row(b)
      accum_row(i, b)
      start_row(i + NBUF, b)
    return 0

  lax.fori_loop(0, (bpw - NBUF) // NBUF, gbody, 0)
  for b in range(NBUF):
    wait_row(b)
    accum_row(bpw - NBUF + b, b)

  pltpu.sync_copy(out_v, out_hbm.at[pl.ds(base, bpw)])


def _pool(ids_mapped, table_flat):
  batch = ids_mapped.shape[0]
  bpw = batch // NW
  mesh = plsc.VectorSubcoreMesh(core_axis_name="c", subcore_axis_name="s")
  k = functools.partial(
      pl.kernel,
      out_type=jax.ShapeDtypeStruct((batch, DIM), jnp.float32),
      mesh=mesh,
      scratch_types=[
          pltpu.VMEM((bpw, SEQ), jnp.int32),
          pltpu.VMEM((NBUF, SEQ, DIM), jnp.float32),
          pltpu.VMEM((bpw, DIM), jnp.float32),
      ] + [pltpu.SemaphoreType.DMA] * NBUF,
      compiler_params=pltpu.CompilerParams(use_tc_tiling_on_sc=False),
  )(_pool_body)
  return k(ids_mapped, table_flat)


def _mlp_body(x_ref, w1_ref, b1_ref, w2_ref, b2_ref, out_ref):
  x = x_ref[:]
  h = jnp.tanh(
      lax.dot_general(x, w1_ref[:], (((1,), (1,)), ((), ())),
                      preferred_element_type=jnp.float32) + b1_ref[:])
  out_ref[:] = lax.dot_general(
      h, w2_ref[:], (((1,), (1,)), ((), ())),
      preferred_element_type=jnp.float32) + b2_ref[:]


def _mlp(pooled, W1, b1, W2, b2):
  batch = pooled.shape[0]
  blk = 1024
  return pl.pallas_call(
      _mlp_body,
      grid=(batch // blk,),
      in_specs=[
          pl.BlockSpec((blk, DIM), lambda i: (i, 0)),
          pl.BlockSpec((DIM, DIM), lambda i: (0, 0)),
          pl.BlockSpec((1, DIM), lambda i: (0, 0)),
          pl.BlockSpec((NUM_CLASSES, DIM), lambda i: (0, 0)),
          pl.BlockSpec((1, NUM_CLASSES), lambda i: (0, 0)),
      ],
      out_specs=pl.BlockSpec((blk, NUM_CLASSES), lambda i: (i, 0)),
      out_shape=jax.ShapeDtypeStruct((batch, NUM_CLASSES), jnp.float32),
  )(pooled, W1, b1.reshape(1, DIM), W2, b2.reshape(1, NUM_CLASSES))


def kernel(word_ids, table, W1, b1, W2, b2):
  ids = word_ids.astype(jnp.int32)
  ids_mapped = ((ids // _DCOL) * _DCOL + 2 * (ids % _DROW)
                + ((ids // _DROW) & 1))
  table2 = _detile(table)
  table_flat = table2.reshape(table2.shape[0] * 2, DIM)
  pooled = _pool(ids_mapped, table_flat)
  return _mlp(pooled, W1, b1, W2, b2)


# speedup vs baseline: 2.6358x; 1.1134x over previous
import functools

import jax
import jax.numpy as jnp
from jax import lax
from jax.experimental import pallas as pl
from jax.experimental.pallas import tpu as pltpu
from jax.experimental.pallas import tpu_sc as plsc

DIM = 64
SEQ = 200
NUM_CLASSES = 128
NC = 2
NS = 16
NW = NC * NS

_C0, _C1 = 104, 96
NBUF = 4

_DCOL = 16384
_DROW = _DCOL // 2


def _detile_body(x_ref, eye_ref, o_ref):
  x = x_ref[:].astype(jnp.bfloat16)
  eye = eye_ref[:]
  o_ref[:, 0:DIM] = lax.dot_general(
      x[:, 0:_DROW], eye, (((0,), (0,)), ((), ())),
      preferred_element_type=jnp.float32)
  o_ref[:, DIM:2 * DIM] = lax.dot_general(
      x[:, _DROW:_DCOL], eye, (((0,), (0,)), ((), ())),
      preferred_element_type=jnp.float32)


def _detile(table):
  vocab = table.shape[0]
  grid = (vocab + _DCOL - 1) // _DCOL
  return pl.pallas_call(
      _detile_body,
      grid=(grid,),
      in_specs=[
          pl.BlockSpec((DIM, _DCOL), lambda i: (0, i)),
          pl.BlockSpec((DIM, DIM), lambda i: (0, 0)),
      ],
      out_specs=pl.BlockSpec((_DROW, 2 * DIM), lambda i: (i, 0)),
      out_shape=jax.ShapeDtypeStruct((grid * _DROW, 2 * DIM), jnp.float32),
  )(table.T, jnp.eye(DIM, dtype=jnp.bfloat16))


def _pool_body(ids_hbm, table_hbm, out_hbm, idx_v, rows_v, out_v, *sems):
  batch = out_hbm.shape[0]
  bpw = batch // NW
  wid = lax.axis_index("s") * NC + lax.axis_index("c")
  base = wid * bpw

  pltpu.sync_copy(ids_hbm.at[pl.ds(base, bpw)], idx_v)

  def start_row(i, b):
    pltpu.make_async_copy(
        table_hbm.at[idx_v.at[i, pl.ds(0, _C0)]],
        rows_v.at[b, pl.ds(0, _C0)], sems[b]).start()
    pltpu.make_async_copy(
        table_hbm.at[idx_v.at[i, pl.ds(_C0, _C1)]],
        rows_v.at[b, pl.ds(_C0, _C1)], sems[b]).start()

  def wait_row(b):
    pltpu.make_async_copy(table_hbm.at[pl.ds(0, SEQ)],
                          rows_v.at[b], sems[b]).wait()

  def accum_row(i, b):
    def jbody(jj, carry):
      a = list(carry)
      j = jj * 4
      for u in range(4):
        for k in range(4):
          a[k] = a[k] + rows_v[b, j + u, pl.ds(16 * k, 16)]
      return tuple(a)
    acc = lax.fori_loop(
        0, SEQ // 4, jbody,
        tuple(jnp.zeros((16,), jnp.float32) for _ in range(4)))
    for k in range(4):
      out_v[i, pl.ds(16 * k, 16)] = acc[k]

  for b in range(NBUF):
    start_row(b, b)

  def gbody(t, _):
    for b in range(NBUF):
      i = t * NBUF + b
      wait_row(b)
      accum_row(i, b)
      start_row(i + NBUF, b)
    return 0

  lax.fori_loop(0, (bpw - NBUF) // NBUF, gbody, 0)
  for b in range(NBUF):
    wait_row(b)
    accum_row(bpw - NBUF + b, b)

  pltpu.sync_copy(out_v, out_hbm.at[pl.ds(base, bpw)])


def _pool(ids_mapped, table_flat):
  batch = ids_mapped.shape[0]
  bpw = batch // NW
  mesh = plsc.VectorSubcoreMesh(core_axis_name="c", subcore_axis_name="s")
  k = functools.partial(
      pl.kernel,
      out_type=jax.ShapeDtypeStruct((batch, DIM), jnp.float32),
      mesh=mesh,
      scratch_types=[
          pltpu.VMEM((bpw, SEQ), jnp.int32),
          pltpu.VMEM((NBUF, SEQ, DIM), jnp.float32),
          pltpu.VMEM((bpw, DIM), jnp.float32),
      ] + [pltpu.SemaphoreType.DMA] * NBUF,
      compiler_params=pltpu.CompilerParams(use_tc_tiling_on_sc=False),
  )(_pool_body)
  return k(ids_mapped, table_flat)


def _mlp_body(x_ref, w1_ref, b1_ref, w2_ref, b2_ref, out_ref):
  x = x_ref[:]
  h = jnp.tanh(
      lax.dot_general(x, w1_ref[:], (((1,), (1,)), ((), ())),
                      preferred_element_type=jnp.float32) + b1_ref[:])
  out_ref[:] = lax.dot_general(
      h, w2_ref[:], (((1,), (1,)), ((), ())),
      preferred_element_type=jnp.float32) + b2_ref[:]


def _mlp(pooled, W1, b1, W2, b2):
  batch = pooled.shape[0]
  blk = 1024
  return pl.pallas_call(
      _mlp_body,
      grid=(batch // blk,),
      in_specs=[
          pl.BlockSpec((blk, DIM), lambda i: (i, 0)),
          pl.BlockSpec((DIM, DIM), lambda i: (0, 0)),
          pl.BlockSpec((1, DIM), lambda i: (0, 0)),
          pl.BlockSpec((NUM_CLASSES, DIM), lambda i: (0, 0)),
          pl.BlockSpec((1, NUM_CLASSES), lambda i: (0, 0)),
      ],
      out_specs=pl.BlockSpec((blk, NUM_CLASSES), lambda i: (i, 0)),
      out_shape=jax.ShapeDtypeStruct((batch, NUM_CLASSES), jnp.float32),
  )(pooled, W1, b1.reshape(1, DIM), W2, b2.reshape(1, NUM_CLASSES))


def kernel(word_ids, table, W1, b1, W2, b2):
  ids = word_ids.astype(jnp.int32)
  ids_mapped = ((ids // _DCOL) * _DCOL + 2 * (ids % _DROW)
                + ((ids // _DROW) & 1))
  table2 = _detile(table)
  table_flat = table2.reshape(table2.shape[0] * 2, DIM)
  pooled = _pool(ids_mapped, table_flat)
  return _mlp(pooled, W1, b1, W2, b2)


# speedup vs baseline: 2.7988x; 1.0618x over previous
import functools

import jax
import jax.numpy as jnp
from jax import lax
from jax.experimental import pallas as pl
from jax.experimental.pallas import tpu as pltpu
from jax.experimental.pallas import tpu_sc as plsc

DIM = 64
SEQ = 200
NUM_CLASSES = 128
NC = 2
NS = 16
NW = NC * NS

_C0, _C1 = 104, 96
NBUF = 4

_DCOL = 32768
_DROW = _DCOL // 2


def _detile_body(x_ref, eye_ref, o_ref):
  x = x_ref[:].astype(jnp.bfloat16)
  eye = eye_ref[:]
  o_ref[:, 0:DIM] = lax.dot_general(
      x[:, 0:_DROW], eye, (((0,), (0,)), ((), ())),
      preferred_element_type=jnp.float32)
  o_ref[:, DIM:2 * DIM] = lax.dot_general(
      x[:, _DROW:_DCOL], eye, (((0,), (0,)), ((), ())),
      preferred_element_type=jnp.float32)


def _detile(table):
  vocab = table.shape[0]
  grid = (vocab + _DCOL - 1) // _DCOL
  return pl.pallas_call(
      _detile_body,
      grid=(grid,),
      in_specs=[
          pl.BlockSpec((DIM, _DCOL), lambda i: (0, i)),
          pl.BlockSpec((DIM, DIM), lambda i: (0, 0)),
      ],
      out_specs=pl.BlockSpec((_DROW, 2 * DIM), lambda i: (i, 0)),
      out_shape=jax.ShapeDtypeStruct((grid * _DROW, 2 * DIM), jnp.float32),
  )(table.T, jnp.eye(DIM, dtype=jnp.bfloat16))


def _pool_body(ids_hbm, table_hbm, out_hbm, idx_v, rows_v, out_v, *sems):
  batch = out_hbm.shape[0]
  bpw = batch // NW
  wid = lax.axis_index("s") * NC + lax.axis_index("c")
  base = wid * bpw

  pltpu.sync_copy(ids_hbm.at[pl.ds(base, bpw)], idx_v)

  def start_row(i, b):
    pltpu.make_async_copy(
        table_hbm.at[idx_v.at[i, pl.ds(0, _C0)]],
        rows_v.at[b, pl.ds(0, _C0)], sems[b]).start()
    pltpu.make_async_copy(
        table_hbm.at[idx_v.at[i, pl.ds(_C0, _C1)]],
        rows_v.at[b, pl.ds(_C0, _C1)], sems[b]).start()

  def wait_row(b):
    pltpu.make_async_copy(table_hbm.at[pl.ds(0, SEQ)],
                          rows_v.at[b], sems[b]).wait()

  def accum_row(i, b):
    def jbody(jj, carry):
      a = list(carry)
      j = jj * 4
      for u in range(4):
        for k in range(4):
          a[k] = a[k] + rows_v[b, j + u, pl.ds(16 * k, 16)]
      return tuple(a)
    acc = lax.fori_loop(
        0, SEQ // 4, jbody,
        tuple(jnp.zeros((16,), jnp.float32) for _ in range(4)))
    for k in range(4):
      out_v[i, pl.ds(16 * k, 16)] = acc[k]

  for b in range(NBUF):
    start_row(b, b)

  def gbody(t, _):
    for b in range(NBUF):
      i = t * NBUF + b
      wait_row(b)
      accum_row(i, b)
      start_row(i + NBUF, b)
    return 0

  lax.fori_loop(0, (bpw - NBUF) // NBUF, gbody, 0)
  for b in range(NBUF):
    wait_row(b)
    accum_row(bpw - NBUF + b, b)

  pltpu.sync_copy(out_v, out_hbm.at[pl.ds(base, bpw)])


def _pool(ids_mapped, table_flat):
  batch = ids_mapped.shape[0]
  bpw = batch // NW
  mesh = plsc.VectorSubcoreMesh(core_axis_name="c", subcore_axis_name="s")
  k = functools.partial(
      pl.kernel,
      out_type=jax.ShapeDtypeStruct((batch, DIM), jnp.float32),
      mesh=mesh,
      scratch_types=[
          pltpu.VMEM((bpw, SEQ), jnp.int32),
          pltpu.VMEM((NBUF, SEQ, DIM), jnp.float32),
          pltpu.VMEM((bpw, DIM), jnp.float32),
      ] + [pltpu.SemaphoreType.DMA] * NBUF,
      compiler_params=pltpu.CompilerParams(use_tc_tiling_on_sc=False),
  )(_pool_body)
  return k(ids_mapped, table_flat)


def _mlp_body(x_ref, w1_ref, b1_ref, w2_ref, b2_ref, out_ref):
  x = x_ref[:]
  h = jnp.tanh(
      lax.dot_general(x, w1_ref[:], (((1,), (1,)), ((), ())),
                      preferred_element_type=jnp.float32) + b1_ref[:])
  out_ref[:] = lax.dot_general(
      h, w2_ref[:], (((1,), (1,)), ((), ())),
      preferred_element_type=jnp.float32) + b2_ref[:]


def _mlp(pooled, W1, b1, W2, b2):
  batch = pooled.shape[0]
  blk = 1024
  return pl.pallas_call(
      _mlp_body,
      grid=(batch // blk,),
      in_specs=[
          pl.BlockSpec((blk, DIM), lambda i: (i, 0)),
          pl.BlockSpec((DIM, DIM), lambda i: (0, 0)),
          pl.BlockSpec((1, DIM), lambda i: (0, 0)),
          pl.BlockSpec((NUM_CLASSES, DIM), lambda i: (0, 0)),
          pl.BlockSpec((1, NUM_CLASSES), lambda i: (0, 0)),
      ],
      out_specs=pl.BlockSpec((blk, NUM_CLASSES), lambda i: (i, 0)),
      out_shape=jax.ShapeDtypeStruct((batch, NUM_CLASSES), jnp.float32),
  )(pooled, W1, b1.reshape(1, DIM), W2, b2.reshape(1, NUM_CLASSES))


def kernel(word_ids, table, W1, b1, W2, b2):
  ids = word_ids.astype(jnp.int32)
  ids_mapped = ((ids // _DCOL) * _DCOL + 2 * (ids % _DROW)
                + ((ids // _DROW) & 1))
  table2 = _detile(table)
  table_flat = table2.reshape(table2.shape[0] * 2, DIM)
  pooled = _pool(ids_mapped, table_flat)
  return _mlp(pooled, W1, b1, W2, b2)
